# baseline (speedup 1.0000x reference)
import functools

import jax
import jax.numpy as jnp
from jax import lax
from jax.experimental import pallas as pl
from jax.experimental.pallas import tpu as pltpu
from jax.experimental.pallas import tpu_sc as plsc

H = 224
W = 224
C = 192
CH = 14
CW = 14
NB = 1000
NIMG = 8
NPIX_IMG = H * W
NPIX = NIMG * NPIX_IMG
NPQ = CH * CW
OUT_ROW = C * NPQ
NW = 32
NCHUNK = C // 16
NG = 4 * CW


@functools.cache
def _build_kernel():
  mesh = plsc.VectorSubcoreMesh(
      core_axis_name="c", subcore_axis_name="s", num_cores=2, num_subcores=16)

  @functools.partial(
      pl.kernel,
      out_type=jax.ShapeDtypeStruct((NB, OUT_ROW), jnp.float32),
      mesh=mesh,
      scratch_types=[
          pltpu.VMEM((NB * 4,), jnp.float32),
          pltpu.VMEM((NB,), jnp.int32),
          pltpu.VMEM((8 * 16,), jnp.float32),
          pltpu.VMEM((2 * 16,), jnp.int32),
          pltpu.VMEM((2, 64), jnp.int32),
          pltpu.VMEM((2, NG, C), jnp.float32),
          pltpu.VMEM((OUT_ROW,), jnp.float32),
          pltpu.SemaphoreType.DMA,
          pltpu.SemaphoreType.DMA,
      ],
      compiler_params=pltpu.CompilerParams(
          needs_layout_passes=False, use_tc_tiling_on_sc=False),
  )
  def crop_kernel(img_ref, boxes_ref, bidx_ref, out_ref,
                  boxes_v, bidx_v, prmf, prmi, idx_v, gbuf, valt,
                  sem0, sem1):
    wid = lax.axis_index("s") * 2 + lax.axis_index("c")
    pltpu.sync_copy(boxes_ref, boxes_v)
    pltpu.sync_copy(bidx_ref, bidx_v)

    ii = lax.iota(jnp.int32, 16)
    fi = ii.astype(jnp.float32)
    sems = (sem0, sem1)

    def splat(x):
      return jnp.full((16,), x, dtype=jnp.int32)

    def bcastf(ref, r, i):
      return plsc.load_gather(ref, [splat(16 * r + i)])

    def wait(par):
      pltpu.make_async_copy(
          img_ref.at[idx_v.at[par, pl.ds(0, NG)]],
          gbuf.at[par], sems[par]).wait()

    def compute_row(i, par, xlv, vxv):
      ylb = bcastf(prmf, 0, i)
      m = bcastf(prmf, 1, i) * vxv
      wb = ylb * m
      wt = m - wb
      onemx = 1.0 - xlv
      prmf[pl.ds(4 * 16, 16)] = onemx * wt
      prmf[pl.ds(5 * 16, 16)] = xlv * wt
      prmf[pl.ds(6 * 16, 16)] = onemx * wb
      prmf[pl.ds(7 * 16, 16)] = xlv * wb

      @plsc.parallel_loop(0, CW, unroll=7)
      def px_body(j):
        wtl = bcastf(prmf, 4, j)
        wtr = bcastf(prmf, 5, j)
        wbl = bcastf(prmf, 6, j)
        wbr = bcastf(prmf, 7, j)
        qoff = (i * CW + j) * C
        for k in range(NCHUNK):
          off = 16 * k
          v = (gbuf[par, j, pl.ds(off, 16)] * wtl
               + gbuf[par, CW + j, pl.ds(off, 16)] * wtr
               + gbuf[par, 2 * CW + j, pl.ds(off, 16)] * wbl
               + gbuf[par, 3 * CW + j, pl.ds(off, 16)] * wbr)
          valt[pl.ds(qoff + off, 16)] = v

    def run_box(t, _):
      n = wid + NW * t
      y1 = plsc.load_gather(boxes_v, [splat(4 * n + 0)])
      x1 = plsc.load_gather(boxes_v, [splat(4 * n + 1)])
      y2 = plsc.load_gather(boxes_v, [splat(4 * n + 2)])
      x2 = plsc.load_gather(boxes_v, [splat(4 * n + 3)])
      b = plsc.load_gather(bidx_v, [splat(n)])
      hs = (y2 - y1) * jnp.float32(H - 1) / jnp.float32(CH - 1)
      ws = (x2 - x1) * jnp.float32(W - 1) / jnp.float32(CW - 1)
      in_y = y1 * jnp.float32(H - 1) + fi * hs
      in_x = x1 * jnp.float32(W - 1) + fi * ws
      tyv = jnp.minimum(jnp.maximum(in_y.astype(jnp.int32), 0), H - 1)
      byv = jnp.minimum(tyv + 1, H - 1)
      ylv = in_y - tyv.astype(jnp.float32)
      vyv = jnp.where((in_y >= 0.0) & (in_y <= jnp.float32(H - 1)), 1.0, 0.0)
      lxv = jnp.minimum(jnp.maximum(in_x.astype(jnp.int32), 0), W - 1)
      rxv = jnp.minimum(lxv + 1, W - 1)
      xlv = in_x - lxv.astype(jnp.float32)
      vxv = jnp.where((in_x >= 0.0) & (in_x <= jnp.float32(W - 1)), 1.0, 0.0)
      prmf[pl.ds(0, 16)] = ylv.astype(jnp.float32)
      prmf[pl.ds(16, 16)] = vyv.astype(jnp.float32)
      base = b * NPIX_IMG
      prmi[pl.ds(0, 16)] = base + tyv * W
      prmi[pl.ds(16, 16)] = base + byv * W

      def issue(i, par):
        bt = bcastf(prmi, 0, i)
        bb = bcastf(prmi, 1, i)
        idx_v[par, pl.ds(0, 16)] = bt + lxv
        idx_v[par, pl.ds(CW, 16)] = bt + rxv
        idx_v[par, pl.ds(2 * CW, 16)] = bb + lxv
        idx_v[par, pl.ds(3 * CW, 16)] = bb + rxv
        pltpu.make_async_copy(
            img_ref.at[idx_v.at[par, pl.ds(0, NG)]],
            gbuf.at[par], sems[par]).start()

      def row_pair(p, _):
        i = 2 * p

        @pl.when(p == 0)
        def _():
          issue(i, 0)

        issue(i + 1, 1)
        wait(0)
        compute_row(i, 0, xlv, vxv)

        @pl.when(p < (CH // 2 - 1))
        def _():
          issue(i + 2, 0)

        wait(1)
        compute_row(i + 1, 1, xlv, vxv)
        return 0

      lax.fori_loop(0, CH // 2, row_pair, 0)
      pltpu.sync_copy(valt, out_ref.at[n])
      return 0

    nboxes = jnp.where(wid < NB - (NB // NW) * NW, NB // NW + 1, NB // NW)
    lax.fori_loop(0, nboxes, run_box, 0)

  return crop_kernel


_YB = 8
_PB = _YB * W
_NYB = H // _YB


_RPY = W * C // 128


def _transpose_body(in_ref, out_ref):
  for yy in range(_YB):
    xt = in_ref[0, :, yy, :].T
    x2 = xt.reshape(W // 2, 2, C)
    a = x2[:, 0, 0:128]
    b = jnp.concatenate([x2[:, 0, 128:C], x2[:, 1, 0:64]], axis=1)
    c = x2[:, 1, 64:C]
    for t, part in enumerate((a, b, c)):
      out_ref[pl.Slice(yy * _RPY + t, W // 2, 3), :] = part


def _to_channel_last(image):
  return pl.pallas_call(
      _transpose_body,
      grid=(NIMG, _NYB),
      in_specs=[pl.BlockSpec((1, C, _YB, W), lambda b, y: (b, 0, y, 0))],
      out_specs=pl.BlockSpec((_YB * _RPY, 128), lambda b, y: (b * _NYB + y, 0)),
      out_shape=jax.ShapeDtypeStruct((NPIX * C // 128, 128), jnp.float32),
  )(image)


def kernel(image, boxes, box_indices):
  img_t = _to_channel_last(image).reshape(NPIX, C)
  out = _build_kernel()(
      img_t, boxes.reshape(NB * 4), box_indices.astype(jnp.int32))
  return out.reshape(NB, CH, CW, C).transpose(0, 3, 1, 2)


# speedup vs baseline: 1.0288x; 1.0288x over previous
import functools

import jax
import jax.numpy as jnp
from jax import lax
from jax.experimental import pallas as pl
from jax.experimental.pallas import tpu as pltpu
from jax.experimental.pallas import tpu_sc as plsc

H = 224
W = 224
C = 192
CH = 14
CW = 14
NB = 1000
NIMG = 8
NPIX_IMG = H * W
NPIX = NIMG * NPIX_IMG
NPQ = CH * CW
OUT_ROW = C * NPQ
NW = 32
NCHUNK = C // 16
NG = 4 * CW


@functools.cache
def _build_kernel():
  mesh = plsc.VectorSubcoreMesh(
      core_axis_name="c", subcore_axis_name="s", num_cores=2, num_subcores=16)

  @functools.partial(
      pl.kernel,
      out_type=jax.ShapeDtypeStruct((NB, OUT_ROW), jnp.float32),
      mesh=mesh,
      scratch_types=[
          pltpu.VMEM((NB * 4,), jnp.float32),
          pltpu.VMEM((NB,), jnp.int32),
          pltpu.VMEM((8 * 16,), jnp.float32),
          pltpu.VMEM((2 * 16,), jnp.int32),
          pltpu.VMEM((2, 64), jnp.int32),
          pltpu.VMEM((2, NG, C), jnp.float32),
          pltpu.VMEM((2, OUT_ROW), jnp.float32),
          pltpu.SemaphoreType.DMA,
          pltpu.SemaphoreType.DMA,
          pltpu.SemaphoreType.DMA,
      ],
      compiler_params=pltpu.CompilerParams(
          needs_layout_passes=False, use_tc_tiling_on_sc=False),
  )
  def crop_kernel(img_ref, boxes_ref, bidx_ref, out_ref,
                  boxes_v, bidx_v, prmf, prmi, idx_v, gbuf, valt,
                  sem0, sem1, semo):
    wid = lax.axis_index("s") * 2 + lax.axis_index("c")
    pltpu.sync_copy(boxes_ref, boxes_v)
    pltpu.sync_copy(bidx_ref, bidx_v)

    ii = lax.iota(jnp.int32, 16)
    fi = ii.astype(jnp.float32)
    sems = (sem0, sem1)

    def splat(x):
      return jnp.full((16,), x, dtype=jnp.int32)

    def bcastf(ref, r, i):
      return plsc.load_gather(ref, [splat(16 * r + i)])

    def wait(par):
      pltpu.make_async_copy(
          img_ref.at[idx_v.at[par, pl.ds(0, NG)]],
          gbuf.at[par], sems[par]).wait()

    def compute_row(i, par, vpar, xlv, vxv):
      ylb = bcastf(prmf, 0, i)
      m = bcastf(prmf, 1, i) * vxv
      wb = ylb * m
      wt = m - wb
      onemx = 1.0 - xlv
      prmf[pl.ds(4 * 16, 16)] = onemx * wt
      prmf[pl.ds(5 * 16, 16)] = xlv * wt
      prmf[pl.ds(6 * 16, 16)] = onemx * wb
      prmf[pl.ds(7 * 16, 16)] = xlv * wb

      @plsc.parallel_loop(0, CW, unroll=2)
      def px_body(j):
        wtl = bcastf(prmf, 4, j)
        wtr = bcastf(prmf, 5, j)
        wbl = bcastf(prmf, 6, j)
        wbr = bcastf(prmf, 7, j)
        qoff = (i * CW + j) * C
        for k in range(NCHUNK):
          off = 16 * k
          v = (gbuf[par, j, pl.ds(off, 16)] * wtl
               + gbuf[par, CW + j, pl.ds(off, 16)] * wtr
               + gbuf[par, 2 * CW + j, pl.ds(off, 16)] * wbl
               + gbuf[par, 3 * CW + j, pl.ds(off, 16)] * wbr)
          valt[vpar, pl.ds(qoff + off, 16)] = v

    def run_box(t, _):
      n = wid + NW * t
      vpar = lax.rem(t, 2)

      @pl.when(t >= 2)
      def _():
        pltpu.make_async_copy(valt.at[0], out_ref.at[0], semo).wait()
      y1 = plsc.load_gather(boxes_v, [splat(4 * n + 0)])
      x1 = plsc.load_gather(boxes_v, [splat(4 * n + 1)])
      y2 = plsc.load_gather(boxes_v, [splat(4 * n + 2)])
      x2 = plsc.load_gather(boxes_v, [splat(4 * n + 3)])
      b = plsc.load_gather(bidx_v, [splat(n)])
      hs = (y2 - y1) * jnp.float32(H - 1) / jnp.float32(CH - 1)
      ws = (x2 - x1) * jnp.float32(W - 1) / jnp.float32(CW - 1)
      in_y = y1 * jnp.float32(H - 1) + fi * hs
      in_x = x1 * jnp.float32(W - 1) + fi * ws
      tyv = jnp.minimum(jnp.maximum(in_y.astype(jnp.int32), 0), H - 1)
      byv = jnp.minimum(tyv + 1, H - 1)
      ylv = in_y - tyv.astype(jnp.float32)
      vyv = jnp.where((in_y >= 0.0) & (in_y <= jnp.float32(H - 1)), 1.0, 0.0)
      lxv = jnp.minimum(jnp.maximum(in_x.astype(jnp.int32), 0), W - 1)
      rxv = jnp.minimum(lxv + 1, W - 1)
      xlv = in_x - lxv.astype(jnp.float32)
      vxv = jnp.where((in_x >= 0.0) & (in_x <= jnp.float32(W - 1)), 1.0, 0.0)
      prmf[pl.ds(0, 16)] = ylv.astype(jnp.float32)
      prmf[pl.ds(16, 16)] = vyv.astype(jnp.float32)
      base = b * NPIX_IMG
      prmi[pl.ds(0, 16)] = base + tyv * W
      prmi[pl.ds(16, 16)] = base + byv * W

      def issue(i, par):
        bt = bcastf(prmi, 0, i)
        bb = bcastf(prmi, 1, i)
        idx_v[par, pl.ds(0, 16)] = bt + lxv
        idx_v[par, pl.ds(CW, 16)] = bt + rxv
        idx_v[par, pl.ds(2 * CW, 16)] = bb + lxv
        idx_v[par, pl.ds(3 * CW, 16)] = bb + rxv
        pltpu.make_async_copy(
            img_ref.at[idx_v.at[par, pl.ds(0, NG)]],
            gbuf.at[par], sems[par]).start()

      def row_pair(p, _):
        i = 2 * p

        @pl.when(p == 0)
        def _():
          issue(i, 0)

        issue(i + 1, 1)
        wait(0)
        compute_row(i, 0, vpar, xlv, vxv)

        @pl.when(p < (CH // 2 - 1))
        def _():
          issue(i + 2, 0)

        wait(1)
        compute_row(i + 1, 1, vpar, xlv, vxv)
        return 0

      lax.fori_loop(0, CH // 2, row_pair, 0)
      pltpu.make_async_copy(valt.at[vpar], out_ref.at[n], semo).start()
      return 0

    nboxes = jnp.where(wid < NB - (NB // NW) * NW, NB // NW + 1, NB // NW)
    lax.fori_loop(0, nboxes, run_box, 0)
    pltpu.make_async_copy(valt.at[0], out_ref.at[0], semo).wait()
    pltpu.make_async_copy(valt.at[0], out_ref.at[0], semo).wait()

  return crop_kernel


_YB = 8
_PB = _YB * W
_NYB = H // _YB


_RPY = W * C // 128


def _transpose_body(in_ref, out_ref):
  for yy in range(_YB):
    xt = in_ref[0, :, yy, :].T
    x2 = xt.reshape(W // 2, 2, C)
    a = x2[:, 0, 0:128]
    b = jnp.concatenate([x2[:, 0, 128:C], x2[:, 1, 0:64]], axis=1)
    c = x2[:, 1, 64:C]
    for t, part in enumerate((a, b, c)):
      out_ref[pl.Slice(yy * _RPY + t, W // 2, 3), :] = part


def _to_channel_last(image):
  return pl.pallas_call(
      _transpose_body,
      grid=(NIMG, _NYB),
      in_specs=[pl.BlockSpec((1, C, _YB, W), lambda b, y: (b, 0, y, 0))],
      out_specs=pl.BlockSpec((_YB * _RPY, 128), lambda b, y: (b * _NYB + y, 0)),
      out_shape=jax.ShapeDtypeStruct((NPIX * C // 128, 128), jnp.float32),
  )(image)


def kernel(image, boxes, box_indices):
  img_t = _to_channel_last(image).reshape(NPIX, C)
  out = _build_kernel()(
      img_t, boxes.reshape(NB * 4), box_indices.astype(jnp.int32))
  return out.reshape(NB, CH, CW, C).transpose(0, 3, 1, 2)
